# baseline (speedup 1.0000x reference)
import functools

import jax
import jax.numpy as jnp
from jax import lax
from jax.experimental import pallas as pl
from jax.experimental.pallas import tpu as pltpu
from jax.experimental.pallas import tpu_sc as plsc

_BS = 100
_K = 10
_ROWS_PER_W = 8
_NW_USED = 13


def _adj_kernel(x_ref, tin_ref, tout_ref, ttr_ref, w_ref,
                a_ref, c0_ref, c1_ref, c2_ref, b_ref,
                adj_ref, v_ref, bspl_ref):
    x = x_ref[...]
    alpha = a_ref[0, 0].astype(jnp.float32)
    c0 = c0_ref[0, 0]
    c1 = c1_ref[0, 0]
    c2 = c2_ref[0, 0]
    b = b_ref[0, 0]

    xc = x - jnp.mean(x, axis=1, keepdims=True)
    g = lax.dot_general(xc, xc, (((1,), (1,)), ((), ())),
                        preferred_element_type=jnp.float32)
    v = lax.dot_general(w_ref[...], x, (((1,), (1,)), ((), ())),
                        preferred_element_type=jnp.float32)

    inv_col = lax.rsqrt(jnp.sum(xc * xc, axis=1, keepdims=True))
    inv_row = inv_col.T
    ri = lax.broadcasted_iota(jnp.int32, (_BS, _BS), 0)
    ci = lax.broadcasted_iota(jnp.int32, (_BS, _BS), 1)
    eye = jnp.where(ri == ci, jnp.float32(1.0), jnp.float32(0.0))
    pheno = c0 * tin_ref[...] + c1 * tout_ref[...] + c2 * ttr_ref[...] + eye

    corr = g * inv_col * inv_row
    dist0 = (1.0 - corr) * (1.0 - eye)
    sigma = jnp.mean(dist0)
    inter = jnp.exp(-(dist0 * dist0) / (2.0 * sigma * sigma))
    fea = (inter - eye) * alpha + eye

    adj_ref[...] = fea * pheno
    v_ref[...] = v
    bspl_ref[...] = jnp.full((1, 16), 0.0625, jnp.float32) * b


def _row_threshold(chunks, mask6):
    neg = jnp.float32(-jnp.inf)
    t = jnp.float32(jnp.inf)
    for _ in range(_K):
        ms = [jnp.where(c < t, c, neg) for c in chunks[:6]]
        ms.append(jnp.where((chunks[6] < t) & mask6, chunks[6], neg))
        mv = ms[0]
        for mm in ms[1:]:
            mv = jnp.maximum(mv, mm)
        m = jnp.max(mv)
        ges = [jnp.where(c >= t, 1.0, 0.0) for c in chunks[:6]]
        ges.append(jnp.where((chunks[6] >= t) & mask6, 1.0, 0.0))
        gv = ges[0]
        for gg in ges[1:]:
            gv = gv + gg
        ge = jnp.sum(gv)
        t = jnp.where(ge <= jnp.float32(_K - 1), m, t)
    return t


def _sc_topk_kernel(adj_hbm, v_hbm, bspl_hbm, out_hbm, rows_v, v_v, b_v, out_v):
    wid = lax.axis_index("s") * 2 + lax.axis_index("c")
    base = wid * _ROWS_PER_W
    lane = lax.iota(jnp.int32, 16)
    mask6 = lane >= 12

    def do_rows(nrows):
        pltpu.sync_copy(adj_hbm.at[pl.ds(base, nrows), :],
                        rows_v.at[pl.ds(0, nrows), :])
        pltpu.sync_copy(v_hbm.at[0], v_v)
        pltpu.sync_copy(bspl_hbm.at[0], b_v)
        vc = [v_v[pl.ds(16 * i, 16)] for i in range(6)] + [v_v[pl.ds(84, 16)]]
        b16 = b_v[...]
        outvec = jnp.zeros((16,), jnp.float32)
        for r in range(nrows):
            chunks = ([rows_v[r, pl.ds(16 * i, 16)] for i in range(6)]
                      + [rows_v[r, pl.ds(84, 16)]])
            t = _row_threshold(chunks, mask6)
            acc = b16
            for i in range(6):
                acc = acc + jnp.where(chunks[i] >= t, chunks[i], 0.0) * vc[i]
            acc = acc + jnp.where((chunks[6] >= t) & mask6,
                                  chunks[6], 0.0) * vc[6]
            outvec = jnp.where(lane == r, jnp.full((16,), jnp.sum(acc)), outvec)
        out_v[...] = outvec
        pltpu.sync_copy(out_v.at[pl.ds(0, nrows)],
                        out_hbm.at[pl.ds(base, nrows)])

    @pl.when(wid < _NW_USED - 1)
    def _():
        do_rows(_ROWS_PER_W)

    @pl.when(wid == _NW_USED - 1)
    def _():
        do_rows(_BS - (_NW_USED - 1) * _ROWS_PER_W)


def kernel(x, alpha, test_in_graph, test_out_graph, train_out_graph, k, c0, c1, c2, W, b):
    del k
    a2 = jnp.reshape(jnp.asarray(alpha), (1, 1))
    adj, v, bspl = pl.pallas_call(
        _adj_kernel,
        out_shape=[
            jax.ShapeDtypeStruct((_BS, _BS), jnp.float32),
            jax.ShapeDtypeStruct((1, _BS), jnp.float32),
            jax.ShapeDtypeStruct((1, 16), jnp.float32),
        ],
    )(x, test_in_graph, test_out_graph, train_out_graph, W, a2,
      jnp.reshape(c0, (1, 1)), jnp.reshape(c1, (1, 1)),
      jnp.reshape(c2, (1, 1)), jnp.reshape(b, (1, 1)))

    sc = functools.partial(
        pl.kernel,
        mesh=plsc.VectorSubcoreMesh(core_axis_name="c", subcore_axis_name="s"),
        compiler_params=pltpu.CompilerParams(needs_layout_passes=False),
        out_type=jax.ShapeDtypeStruct((_BS,), jnp.float32),
        scratch_types=[
            pltpu.VMEM((_ROWS_PER_W, _BS), jnp.float32),
            pltpu.VMEM((_BS,), jnp.float32),
            pltpu.VMEM((16,), jnp.float32),
            pltpu.VMEM((16,), jnp.float32),
        ],
    )(_sc_topk_kernel)
    return sc(adj, v, bspl)


# speedup vs baseline: 5.4413x; 5.4413x over previous
import jax
import jax.numpy as jnp
from jax import lax
from jax.experimental import pallas as pl

_BS = 100
_HID = 1024
_K = 10


def _gcn_kernel(x_ref, tin_ref, tout_ref, ttr_ref, w_ref,
                a_ref, c0_ref, c1_ref, c2_ref, b_ref, out_ref):
    x = x_ref[...]
    g0 = lax.dot_general(x, x, (((1,), (1,)), ((), ())),
                         preferred_element_type=jnp.float32)
    v = lax.dot_general(x, w_ref[...], (((1,), (1,)), ((), ())),
                        preferred_element_type=jnp.float32)

    alpha = a_ref[0, 0].astype(jnp.float32)
    c0 = c0_ref[0, 0]
    c1 = c1_ref[0, 0]
    c2 = c2_ref[0, 0]
    b = b_ref[0, 0]

    mu_col = jnp.mean(x, axis=1, keepdims=True)
    mu_row = mu_col.T
    ri = lax.broadcasted_iota(jnp.int32, (_BS, _BS), 0)
    ci = lax.broadcasted_iota(jnp.int32, (_BS, _BS), 1)
    eye = jnp.where(ri == ci, jnp.float32(1.0), jnp.float32(0.0))
    pheno = c0 * tin_ref[...] + c1 * tout_ref[...] + c2 * ttr_ref[...] + eye
    pheno_t = pheno.T

    g = g0 - (jnp.float32(_HID) * mu_col) * mu_row
    d_row = jnp.sum(g * eye, axis=0, keepdims=True)
    inv_row = lax.rsqrt(d_row)
    inv_col = inv_row.T
    corr = g * inv_col * inv_row

    dist0 = (1.0 - corr) * (1.0 - eye)
    d2 = dist0 * dist0
    sigma = jnp.mean(dist0)
    inter = jnp.exp(d2 * (jnp.float32(-0.5) / (sigma * sigma)))
    fea = (inter - eye) * alpha + eye

    adj_t = fea * pheno_t

    neg = jnp.float32(-jnp.inf)
    t = jnp.full((1, _BS), jnp.inf, jnp.float32)
    for _ in range(_K):
        lt = adj_t < t
        m = jnp.max(jnp.where(lt, adj_t, neg), axis=0, keepdims=True)
        ge = jnp.sum(jnp.where(lt, 0.0, 1.0), axis=0, keepdims=True)
        t = jnp.where(ge <= jnp.float32(_K - 1), m, t)
    adjm_t = jnp.where(adj_t < t, jnp.float32(0.0), adj_t)

    out = jnp.sum(adjm_t * v, axis=0, keepdims=True) + b
    out_ref[...] = out


def kernel(x, alpha, test_in_graph, test_out_graph, train_out_graph, k, c0, c1, c2, W, b):
    del k
    a2 = jnp.reshape(jnp.asarray(alpha), (1, 1))
    out = pl.pallas_call(
        _gcn_kernel,
        out_shape=jax.ShapeDtypeStruct((1, _BS), jnp.float32),
    )(x, test_in_graph, test_out_graph, train_out_graph, W, a2,
      jnp.reshape(c0, (1, 1)), jnp.reshape(c1, (1, 1)),
      jnp.reshape(c2, (1, 1)), jnp.reshape(b, (1, 1)))
    return out[0]
